# Initial kernel scaffold; baseline (speedup 1.0000x reference)
import jax
import jax.numpy as jnp
from jax.experimental import pallas as pl


def kernel(x, edge_index, edge_type, Wr1, Ws1, b1, Wr2, Ws2, b2, Wi1, bi1, Wi2, bi2, Wj1, bj1, Wj2, bj2, Wf1, bf1, Wf2, bf2):
    raise NotImplementedError("write your pallas kernel here")


# speedup vs baseline: 11.9143x; 11.9143x over previous
import functools

import jax
import jax.numpy as jnp
from jax import lax
from jax.experimental import pallas as pl
from jax.experimental.pallas import tpu as pltpu
from jax.experimental.pallas import tpu_sc as plsc

N = 50000
E = 800000
R = 4
D_IN = 16
D_H = 32

NC = 2
NS = 16
NW = NC * NS
CHUNK = 128
E_PAD = 819200
E_HALF = E_PAD // NC
E_TILE = E_HALF // NS
CH_PER_TILE = E_TILE // CHUNK
N_ACC = 51200
ZROWS = N_ACC // NS
ZCH = ZROWS // CHUNK

BN = 2000
GRID_N = N // BN


def _sc_body(keys_hbm, dst_hbm, table_hbm, zeros_hbm, out_hbm,
             key_v, dst_v, rows_v, agg_sh, sem):
  c = lax.axis_index("c")
  s = lax.axis_index("s")

  pltpu.sync_copy(zeros_hbm, rows_v)
  zbase = s * ZROWS

  def zero_body(i, carry):
    r0 = pl.multiple_of(zbase + i * CHUNK, CHUNK)
    pltpu.sync_copy(rows_v, agg_sh.at[pl.ds(r0, CHUNK)])
    return carry

  lax.fori_loop(0, ZCH, zero_body, 0)
  plsc.subcore_barrier()

  ebase = c * E_HALF + s * E_TILE

  def edge_body(j, carry):
    base = pl.multiple_of(ebase + j * CHUNK, CHUNK)
    pltpu.sync_copy(keys_hbm.at[pl.ds(base, CHUNK)], key_v)
    pltpu.sync_copy(dst_hbm.at[pl.ds(base, CHUNK)], dst_v)
    pltpu.async_copy(table_hbm.at[key_v], rows_v, sem).wait()
    pltpu.sync_copy(rows_v, agg_sh.at[dst_v], add=True)
    return carry

  lax.fori_loop(0, CH_PER_TILE, edge_body, 0)
  plsc.subcore_barrier()

  obase = c * N_ACC + s * ZROWS

  def out_body(i, carry):
    r0 = pl.multiple_of(zbase + i * CHUNK, CHUNK)
    o0 = pl.multiple_of(obase + i * CHUNK, CHUNK)
    pltpu.sync_copy(agg_sh.at[pl.ds(r0, CHUNK)], rows_v)
    pltpu.sync_copy(rows_v, out_hbm.at[pl.ds(o0, CHUNK)])
    return carry

  lax.fori_loop(0, ZCH, out_body, 0)


_sc_segsum = pl.kernel(
    _sc_body,
    out_type=jax.ShapeDtypeStruct((NC * N_ACC, D_H), jnp.float32),
    mesh=plsc.VectorSubcoreMesh(core_axis_name="c", subcore_axis_name="s"),
    scratch_types=[
        pltpu.VMEM((CHUNK,), jnp.int32),
        pltpu.VMEM((CHUNK,), jnp.int32),
        pltpu.VMEM((CHUNK, D_H), jnp.float32),
        pltpu.VMEM_SHARED((N_ACC, D_H), jnp.float32),
        pltpu.SemaphoreType.DMA,
    ],
    compiler_params=pltpu.CompilerParams(use_tc_tiling_on_sc=False),
)


def _keys_body(et_ref, src_ref, keys_ref):
  keys_ref[...] = et_ref[...] * N + src_ref[...]


def _prep_keys(et2d, src2d):
  rows = et2d.shape[0]
  blk = rows // 8
  return pl.pallas_call(
      _keys_body,
      grid=(8,),
      in_specs=[
          pl.BlockSpec((blk, 128), lambda i: (i, 0)),
          pl.BlockSpec((blk, 128), lambda i: (i, 0)),
      ],
      out_specs=pl.BlockSpec((blk, 128), lambda i: (i, 0)),
      out_shape=jax.ShapeDtypeStruct((rows, 128), jnp.int32),
  )(et2d, src2d)


def _transform1_body(x_ref, wr_ref, t_ref):
  xb = x_ref[...]
  for r in range(R):
    t_ref[r] = jnp.dot(xb, wr_ref[r], preferred_element_type=jnp.float32)


def _transform1(x, Wr1):
  return pl.pallas_call(
      _transform1_body,
      grid=(GRID_N,),
      in_specs=[
          pl.BlockSpec((BN, D_IN), lambda i: (i, 0)),
          pl.BlockSpec((R, D_IN, D_H), lambda i: (0, 0, 0)),
      ],
      out_specs=pl.BlockSpec((R, BN, D_H), lambda i: (0, i, 0)),
      out_shape=jax.ShapeDtypeStruct((R, N, D_H), jnp.float32),
  )(x, Wr1)


def _mid_body(a_ref, b_ref, x_ref, ws_ref, bias_ref, wr_ref, h_ref, t_ref):
  agg = a_ref[0] + b_ref[0]
  h = jnp.tanh(agg + jnp.dot(x_ref[...], ws_ref[...],
                             preferred_element_type=jnp.float32)
               + bias_ref[...])
  h_ref[...] = h
  for r in range(R):
    t_ref[r] = jnp.dot(h, wr_ref[r], preferred_element_type=jnp.float32)


def _layer_mid(aggp, x, Ws1, b1, Wr2):
  return pl.pallas_call(
      _mid_body,
      grid=(GRID_N,),
      in_specs=[
          pl.BlockSpec((1, BN, D_H), lambda i: (0, i, 0)),
          pl.BlockSpec((1, BN, D_H), lambda i: (1, i, 0)),
          pl.BlockSpec((BN, D_IN), lambda i: (i, 0)),
          pl.BlockSpec((D_IN, D_H), lambda i: (0, 0)),
          pl.BlockSpec((1, D_H), lambda i: (0, 0)),
          pl.BlockSpec((R, D_H, D_H), lambda i: (0, 0, 0)),
      ],
      out_specs=[
          pl.BlockSpec((BN, D_H), lambda i: (i, 0)),
          pl.BlockSpec((R, BN, D_H), lambda i: (0, i, 0)),
      ],
      out_shape=[
          jax.ShapeDtypeStruct((N, D_H), jnp.float32),
          jax.ShapeDtypeStruct((R, N, D_H), jnp.float32),
      ],
  )(aggp, aggp, x, Ws1, b1, Wr2)


def _head_body(a_ref, b_ref, h1_ref, x_ref, ws_ref, bias_ref,
               wi1a_ref, wi1b_ref, bi1_ref, wi2_ref, bi2_ref,
               wj1a_ref, wj1b_ref, bj1_ref, wj2_ref, bj2_ref,
               wf1_ref, bf1_ref, wf2_ref, bf2_ref,
               out_ref, acc_ref):
  i = pl.program_id(0)

  @pl.when(i == 0)
  def _():
    acc_ref[...] = jnp.zeros_like(acc_ref)

  agg = a_ref[0] + b_ref[0]
  h1 = h1_ref[...]
  xb = x_ref[...]
  h2 = jnp.tanh(agg + jnp.dot(h1, ws_ref[...],
                              preferred_element_type=jnp.float32)
                + bias_ref[...])
  u = jnp.maximum(
      jnp.dot(xb, wi1a_ref[...], preferred_element_type=jnp.float32)
      + jnp.dot(h2, wi1b_ref[...], preferred_element_type=jnp.float32)
      + bi1_ref[...], 0.0)
  io = jax.nn.sigmoid(jnp.dot(u, wi2_ref[...],
                              preferred_element_type=jnp.float32)
                      + bi2_ref[...])
  v = jnp.maximum(
      jnp.dot(xb, wj1a_ref[...], preferred_element_type=jnp.float32)
      + jnp.dot(h2, wj1b_ref[...], preferred_element_type=jnp.float32)
      + bj1_ref[...], 0.0)
  jo = jnp.tanh(jnp.dot(v, wj2_ref[...], preferred_element_type=jnp.float32)
                + bj2_ref[...])
  p = jnp.sum(io * jo, axis=0, keepdims=True)
  acc_ref[0:1, 0:D_H] = acc_ref[0:1, 0:D_H] + p

  @pl.when(i == GRID_N - 1)
  def _():
    g = jnp.tanh(acc_ref[0:1, 0:D_H])
    f = jnp.maximum(
        jnp.dot(g, wf1_ref[...], preferred_element_type=jnp.float32)
        + bf1_ref[...], 0.0)
    out_ref[...] = (jnp.dot(f, wf2_ref[...],
                            preferred_element_type=jnp.float32)
                    + bf2_ref[...])


def _head(aggp, h1, x, Ws2, b2, Wi1a, Wi1b, bi1, Wi2, bi2,
          Wj1a, Wj1b, bj1, Wj2, bj2, Wf1, bf1, Wf2, bf2):
  full = lambda shape: pl.BlockSpec(shape, lambda i: tuple(0 for _ in shape))
  return pl.pallas_call(
      _head_body,
      grid=(GRID_N,),
      in_specs=[
          pl.BlockSpec((1, BN, D_H), lambda i: (0, i, 0)),
          pl.BlockSpec((1, BN, D_H), lambda i: (1, i, 0)),
          pl.BlockSpec((BN, D_H), lambda i: (i, 0)),
          pl.BlockSpec((BN, D_IN), lambda i: (i, 0)),
          full((D_H, D_H)), full((1, D_H)),
          full((D_IN, 64)), full((D_H, 64)), full((1, 64)),
          full((64, D_H)), full((1, D_H)),
          full((D_IN, 64)), full((D_H, 64)), full((1, 64)),
          full((64, D_H)), full((1, D_H)),
          full((D_H, 64)), full((1, 64)), full((64, 1)), full((1, 1)),
      ],
      out_specs=pl.BlockSpec((1, 1), lambda i: (0, 0)),
      out_shape=jax.ShapeDtypeStruct((1, 1), jnp.float32),
      scratch_shapes=[pltpu.VMEM((8, 128), jnp.float32)],
  )(aggp, aggp, h1, x, Ws2, b2, Wi1a, Wi1b, bi1, Wi2, bi2,
    Wj1a, Wj1b, bj1, Wj2, bj2, Wf1, bf1, Wf2, bf2)


def kernel(x, edge_index, edge_type, Wr1, Ws1, b1, Wr2, Ws2, b2,
           Wi1, bi1, Wi2, bi2, Wj1, bj1, Wj2, bj2, Wf1, bf1, Wf2, bf2):
  src = edge_index[0]
  dst = edge_index[1]
  pad = E_PAD - E
  src_p = jnp.concatenate([src, jnp.zeros((pad,), jnp.int32)])
  et_p = jnp.concatenate([edge_type, jnp.zeros((pad,), jnp.int32)])
  dst_p = jnp.concatenate([dst, jnp.full((pad,), N, jnp.int32)])

  keys = _prep_keys(et_p.reshape(-1, 128), src_p.reshape(-1, 128)).reshape(-1)
  zeros128 = jnp.zeros((CHUNK, D_H), jnp.float32)

  t1 = _transform1(x, Wr1)
  aggp1 = _sc_segsum(keys, dst_p, t1.reshape(R * N, D_H), zeros128)
  h1, t2 = _layer_mid(aggp1.reshape(NC, N_ACC, D_H), x, Ws1,
                      b1.reshape(1, D_H), Wr2)
  aggp2 = _sc_segsum(keys, dst_p, t2.reshape(R * N, D_H), zeros128)
  out = _head(aggp2.reshape(NC, N_ACC, D_H), h1, x, Ws2,
              b2.reshape(1, D_H),
              Wi1[:D_IN], Wi1[D_IN:], bi1.reshape(1, 64),
              Wi2, bi2.reshape(1, D_H),
              Wj1[:D_IN], Wj1[D_IN:], bj1.reshape(1, 64),
              Wj2, bj2.reshape(1, D_H),
              Wf1, bf1.reshape(1, 64), Wf2, bf2.reshape(1, 1))
  return out.reshape(1)


# speedup vs baseline: 16.6929x; 1.4011x over previous
import jax
import jax.numpy as jnp
from jax import lax
from jax.experimental import pallas as pl
from jax.experimental.pallas import tpu as pltpu
from jax.experimental.pallas import tpu_sc as plsc

N = 50000
E = 800000
R = 4
D_IN = 16
D_H = 32

NC = 2
NS = 16
NW = NC * NS
CHUNK = 128
E_PAD = 819200
CH_TOTAL = E_PAD // CHUNK
CH_PER_TILE = CH_TOTAL // NW
K = 4
SG = 20
N_ACC = 51200
ZROWS = N_ACC // NS
ZCH = ZROWS // CHUNK

EROWS_IN = E // CHUNK
PB_IN = EROWS_IN
PB_OUT = CH_TOTAL
PGRID = 1

BN = 2000
GRID_N = N // BN


def _sc_body(keys_hbm, dst_hbm, table_hbm, zeros_hbm, out_hbm,
             keysb, dstb, rows_v, agg_sh, gsem, ssem, zsem):
  c = lax.axis_index("c")
  s = lax.axis_index("s")
  tid = c * NS + s
  zbase = s * ZROWS

  pltpu.sync_copy(zeros_hbm, rows_v.at[0])

  def zgroup(i, carry):
    ds = []
    for b in range(K):
      r0 = pl.multiple_of(zbase + (i * K + b) * CHUNK, CHUNK)
      ds.append(pltpu.async_copy(rows_v.at[0], agg_sh.at[pl.ds(r0, CHUNK)],
                                 zsem))
    for d in ds:
      d.wait()
    return carry

  lax.fori_loop(0, ZCH // K, zgroup, 0)
  r0 = pl.multiple_of(zbase + (ZCH // K) * K * CHUNK, CHUNK)
  pltpu.sync_copy(rows_v.at[0], agg_sh.at[pl.ds(r0, CHUNK)])

  cbase = pl.multiple_of(tid * CH_PER_TILE, 8)
  plsc.subcore_barrier()

  def sg_body(sg, carry):
    i0 = pl.multiple_of(cbase + sg * SG, 8)
    pltpu.sync_copy(keys_hbm.at[pl.ds(i0, SG)], keysb)
    pltpu.sync_copy(dst_hbm.at[pl.ds(i0, SG)], dstb)

    def group_body(g, carry2):
      gds = [pltpu.async_copy(table_hbm.at[keysb.at[g * K + b]],
                              rows_v.at[b], gsem) for b in range(K)]
      for d in gds:
        d.wait()
      sds = [pltpu.async_copy(rows_v.at[b], agg_sh.at[dstb.at[g * K + b]],
                              ssem, add=True) for b in range(K)]
      for d in sds:
        d.wait()
      return carry2

    lax.fori_loop(0, SG // K, group_body, 0)
    return carry

  lax.fori_loop(0, CH_PER_TILE // SG, sg_body, 0)
  plsc.subcore_barrier()

  obase = c * N_ACC + s * ZROWS

  def ogroup(i, carry):
    ds = []
    for b in range(K):
      r0 = pl.multiple_of(zbase + (i * K + b) * CHUNK, CHUNK)
      ds.append(pltpu.async_copy(agg_sh.at[pl.ds(r0, CHUNK)], rows_v.at[b],
                                 zsem))
    for d in ds:
      d.wait()
    ds = []
    for b in range(K):
      o0 = pl.multiple_of(obase + (i * K + b) * CHUNK, CHUNK)
      ds.append(pltpu.async_copy(rows_v.at[b], out_hbm.at[pl.ds(o0, CHUNK)],
                                 zsem))
    for d in ds:
      d.wait()
    return carry

  lax.fori_loop(0, ZCH // K, ogroup, 0)
  r0 = pl.multiple_of(zbase + (ZCH // K) * K * CHUNK, CHUNK)
  o0 = pl.multiple_of(obase + (ZCH // K) * K * CHUNK, CHUNK)
  pltpu.sync_copy(agg_sh.at[pl.ds(r0, CHUNK)], rows_v.at[0])
  pltpu.sync_copy(rows_v.at[0], out_hbm.at[pl.ds(o0, CHUNK)])


_sc_segsum = pl.kernel(
    _sc_body,
    out_type=jax.ShapeDtypeStruct((NC * N_ACC, D_H), jnp.float32),
    mesh=plsc.VectorSubcoreMesh(core_axis_name="c", subcore_axis_name="s"),
    scratch_types=[
        pltpu.VMEM((SG, CHUNK), jnp.int32),
        pltpu.VMEM((SG, CHUNK), jnp.int32),
        pltpu.VMEM((K, CHUNK, D_H), jnp.float32),
        pltpu.VMEM_SHARED((N_ACC, D_H), jnp.float32),
        pltpu.SemaphoreType.DMA,
        pltpu.SemaphoreType.DMA,
        pltpu.SemaphoreType.DMA,
    ],
    compiler_params=pltpu.CompilerParams(use_tc_tiling_on_sc=False),
)


def _prep_body(src_ref, dst_ref, et_ref, keys_ref, dstp_ref):
  keys = et_ref[...] * N + src_ref[...]
  pad_k = jnp.zeros((PB_OUT - PB_IN, CHUNK), jnp.int32)
  flat = jax.lax.broadcasted_iota(jnp.int32, (PB_OUT - PB_IN, CHUNK), 0) \
      * CHUNK + jax.lax.broadcasted_iota(
          jnp.int32, (PB_OUT - PB_IN, CHUNK), 1)
  pad_d = N + (flat % (N_ACC - N))
  keys_ref[...] = jnp.concatenate([keys, pad_k], axis=0)
  dstp_ref[...] = jnp.concatenate([dst_ref[...], pad_d], axis=0)


def _prep_edges(src2d, dst2d, et2d):
  return pl.pallas_call(
      _prep_body,
      grid=(PGRID,),
      in_specs=[
          pl.BlockSpec((PB_IN, CHUNK), lambda i: (i, 0)),
          pl.BlockSpec((PB_IN, CHUNK), lambda i: (i, 0)),
          pl.BlockSpec((PB_IN, CHUNK), lambda i: (i, 0)),
      ],
      out_specs=[
          pl.BlockSpec((PB_OUT, CHUNK), lambda i: (i, 0)),
          pl.BlockSpec((PB_OUT, CHUNK), lambda i: (i, 0)),
      ],
      out_shape=[
          jax.ShapeDtypeStruct((CH_TOTAL, CHUNK), jnp.int32),
          jax.ShapeDtypeStruct((CH_TOTAL, CHUNK), jnp.int32),
      ],
  )(src2d, dst2d, et2d)


def _transform1_body(x_ref, wr_ref, t_ref):
  xb = x_ref[...]
  for r in range(R):
    t_ref[r] = jnp.dot(xb, wr_ref[r], preferred_element_type=jnp.float32)


def _transform1(x, Wr1):
  return pl.pallas_call(
      _transform1_body,
      grid=(GRID_N,),
      in_specs=[
          pl.BlockSpec((BN, D_IN), lambda i: (i, 0)),
          pl.BlockSpec((R, D_IN, D_H), lambda i: (0, 0, 0)),
      ],
      out_specs=pl.BlockSpec((R, BN, D_H), lambda i: (0, i, 0)),
      out_shape=jax.ShapeDtypeStruct((R, N, D_H), jnp.float32),
  )(x, Wr1)


def _mid_body(a_ref, b_ref, x_ref, ws_ref, bias_ref, wr_ref, h_ref, t_ref):
  agg = a_ref[0] + b_ref[0]
  h = jnp.tanh(agg + jnp.dot(x_ref[...], ws_ref[...],
                             preferred_element_type=jnp.float32)
               + bias_ref[...])
  h_ref[...] = h
  for r in range(R):
    t_ref[r] = jnp.dot(h, wr_ref[r], preferred_element_type=jnp.float32)


def _layer_mid(aggp, x, Ws1, b1, Wr2):
  return pl.pallas_call(
      _mid_body,
      grid=(GRID_N,),
      in_specs=[
          pl.BlockSpec((1, BN, D_H), lambda i: (0, i, 0)),
          pl.BlockSpec((1, BN, D_H), lambda i: (1, i, 0)),
          pl.BlockSpec((BN, D_IN), lambda i: (i, 0)),
          pl.BlockSpec((D_IN, D_H), lambda i: (0, 0)),
          pl.BlockSpec((1, D_H), lambda i: (0, 0)),
          pl.BlockSpec((R, D_H, D_H), lambda i: (0, 0, 0)),
      ],
      out_specs=[
          pl.BlockSpec((BN, D_H), lambda i: (i, 0)),
          pl.BlockSpec((R, BN, D_H), lambda i: (0, i, 0)),
      ],
      out_shape=[
          jax.ShapeDtypeStruct((N, D_H), jnp.float32),
          jax.ShapeDtypeStruct((R, N, D_H), jnp.float32),
      ],
  )(aggp, aggp, x, Ws1, b1, Wr2)


def _head_body(a_ref, b_ref, h1_ref, x_ref, ws_ref, bias_ref,
               wi1a_ref, wi1b_ref, bi1_ref, wi2_ref, bi2_ref,
               wj1a_ref, wj1b_ref, bj1_ref, wj2_ref, bj2_ref,
               wf1_ref, bf1_ref, wf2_ref, bf2_ref,
               out_ref, acc_ref):
  i = pl.program_id(0)

  @pl.when(i == 0)
  def _():
    acc_ref[...] = jnp.zeros_like(acc_ref)

  agg = a_ref[0] + b_ref[0]
  h1 = h1_ref[...]
  xb = x_ref[...]
  h2 = jnp.tanh(agg + jnp.dot(h1, ws_ref[...],
                              preferred_element_type=jnp.float32)
                + bias_ref[...])
  u = jnp.maximum(
      jnp.dot(xb, wi1a_ref[...], preferred_element_type=jnp.float32)
      + jnp.dot(h2, wi1b_ref[...], preferred_element_type=jnp.float32)
      + bi1_ref[...], 0.0)
  io = jax.nn.sigmoid(jnp.dot(u, wi2_ref[...],
                              preferred_element_type=jnp.float32)
                      + bi2_ref[...])
  v = jnp.maximum(
      jnp.dot(xb, wj1a_ref[...], preferred_element_type=jnp.float32)
      + jnp.dot(h2, wj1b_ref[...], preferred_element_type=jnp.float32)
      + bj1_ref[...], 0.0)
  jo = jnp.tanh(jnp.dot(v, wj2_ref[...], preferred_element_type=jnp.float32)
                + bj2_ref[...])
  p = jnp.sum(io * jo, axis=0, keepdims=True)
  acc_ref[0:1, 0:D_H] = acc_ref[0:1, 0:D_H] + p

  @pl.when(i == GRID_N - 1)
  def _():
    g = jnp.tanh(acc_ref[0:1, 0:D_H])
    f = jnp.maximum(
        jnp.dot(g, wf1_ref[...], preferred_element_type=jnp.float32)
        + bf1_ref[...], 0.0)
    out_ref[...] = (jnp.dot(f, wf2_ref[...],
                            preferred_element_type=jnp.float32)
                    + bf2_ref[...])


def _head(aggp, h1, x, Ws2, b2, Wi1a, Wi1b, bi1, Wi2, bi2,
          Wj1a, Wj1b, bj1, Wj2, bj2, Wf1, bf1, Wf2, bf2):
  full = lambda shape: pl.BlockSpec(shape, lambda i: tuple(0 for _ in shape))
  return pl.pallas_call(
      _head_body,
      grid=(GRID_N,),
      in_specs=[
          pl.BlockSpec((1, BN, D_H), lambda i: (0, i, 0)),
          pl.BlockSpec((1, BN, D_H), lambda i: (1, i, 0)),
          pl.BlockSpec((BN, D_H), lambda i: (i, 0)),
          pl.BlockSpec((BN, D_IN), lambda i: (i, 0)),
          full((D_H, D_H)), full((1, D_H)),
          full((D_IN, 64)), full((D_H, 64)), full((1, 64)),
          full((64, D_H)), full((1, D_H)),
          full((D_IN, 64)), full((D_H, 64)), full((1, 64)),
          full((64, D_H)), full((1, D_H)),
          full((D_H, 64)), full((1, 64)), full((64, 1)), full((1, 1)),
      ],
      out_specs=pl.BlockSpec((1, 1), lambda i: (0, 0)),
      out_shape=jax.ShapeDtypeStruct((1, 1), jnp.float32),
      scratch_shapes=[pltpu.VMEM((8, 128), jnp.float32)],
  )(aggp, aggp, h1, x, Ws2, b2, Wi1a, Wi1b, bi1, Wi2, bi2,
    Wj1a, Wj1b, bj1, Wj2, bj2, Wf1, bf1, Wf2, bf2)


def kernel(x, edge_index, edge_type, Wr1, Ws1, b1, Wr2, Ws2, b2,
           Wi1, bi1, Wi2, bi2, Wj1, bj1, Wj2, bj2, Wf1, bf1, Wf2, bf2):
  src2d = edge_index[0].reshape(EROWS_IN, CHUNK)
  dst2d = edge_index[1].reshape(EROWS_IN, CHUNK)
  et2d = edge_type.reshape(EROWS_IN, CHUNK)

  keys2d, dstp2d = _prep_edges(src2d, dst2d, et2d)
  zeros128 = jnp.zeros((CHUNK, D_H), jnp.float32)

  t1 = _transform1(x, Wr1)
  aggp1 = _sc_segsum(keys2d, dstp2d, t1.reshape(R * N, D_H), zeros128)
  h1, t2 = _layer_mid(aggp1.reshape(NC, N_ACC, D_H), x, Ws1,
                      b1.reshape(1, D_H), Wr2)
  aggp2 = _sc_segsum(keys2d, dstp2d, t2.reshape(R * N, D_H), zeros128)
  out = _head(aggp2.reshape(NC, N_ACC, D_H), h1, x, Ws2,
              b2.reshape(1, D_H),
              Wi1[:D_IN], Wi1[D_IN:], bi1.reshape(1, 64),
              Wi2, bi2.reshape(1, D_H),
              Wj1[:D_IN], Wj1[D_IN:], bj1.reshape(1, 64),
              Wj2, bj2.reshape(1, D_H),
              Wf1, bf1.reshape(1, 64), Wf2, bf2.reshape(1, 1))
  return out.reshape(1)


# speedup vs baseline: 16.7948x; 1.0061x over previous
import jax
import jax.numpy as jnp
from jax import lax
from jax.experimental import pallas as pl
from jax.experimental.pallas import tpu as pltpu
from jax.experimental.pallas import tpu_sc as plsc

N = 50000
E = 800000
R = 4
D_IN = 16
D_H = 32

NC = 2
NS = 16
NW = NC * NS
CHUNK = 128
E_PAD = 819200
CH_TOTAL = E_PAD // CHUNK
CH_PER_TILE = CH_TOTAL // NW
K = 5
SG = 20
NSG0 = 10
NSG1 = 10
N_ACC = 51200
ZROWS = N_ACC // NS
ZCH = ZROWS // CHUNK

EROWS_IN = E // CHUNK
PB_IN = EROWS_IN
PB_OUT = CH_TOTAL
PGRID = 1

BN = 2000
GRID_N = N // BN


def _sc_body(keys_hbm, dst_hbm, table_hbm, zeros_hbm, out_hbm,
             keysb, dstb, rows_v, agg_sh, gsem, ssem, zsem):
  c = lax.axis_index("c")
  s = lax.axis_index("s")
  tid = c * NS + s
  zbase = s * ZROWS

  pltpu.sync_copy(zeros_hbm, rows_v.at[0])

  def zgroup(i, carry):
    ds = []
    for b in range(K):
      r0 = pl.multiple_of(zbase + (i * K + b) * CHUNK, CHUNK)
      ds.append(pltpu.async_copy(rows_v.at[0], agg_sh.at[pl.ds(r0, CHUNK)],
                                 zsem))
    for d in ds:
      d.wait()
    return carry

  lax.fori_loop(0, ZCH // K, zgroup, 0)
  r0 = pl.multiple_of(zbase + (ZCH // K) * K * CHUNK, CHUNK)
  pltpu.sync_copy(rows_v.at[0], agg_sh.at[pl.ds(r0, CHUNK)])

  nsg = jnp.where(c == 0, NSG0, NSG1)
  cbase = pl.multiple_of(
      jnp.where(c == 0, s * (NSG0 * SG),
                16 * NSG0 * SG + s * (NSG1 * SG)), 8)
  plsc.subcore_barrier()

  def sg_body(sg, carry):
    i0 = pl.multiple_of(cbase + sg * SG, 8)
    pltpu.sync_copy(keys_hbm.at[pl.ds(i0, SG)], keysb)
    pltpu.sync_copy(dst_hbm.at[pl.ds(i0, SG)], dstb)

    def group_body(g, carry2):
      gds = [pltpu.async_copy(table_hbm.at[keysb.at[g * K + b]],
                              rows_v.at[b], gsem) for b in range(K)]
      for d in gds:
        d.wait()
      sds = [pltpu.async_copy(rows_v.at[b], agg_sh.at[dstb.at[g * K + b]],
                              ssem, add=True) for b in range(K)]
      for d in sds:
        d.wait()
      return carry2

    lax.fori_loop(0, SG // K, group_body, 0)
    return carry

  lax.fori_loop(0, nsg, sg_body, 0)
  plsc.subcore_barrier()

  obase = c * N_ACC + s * ZROWS

  def ogroup(i, carry):
    ds = []
    for b in range(K):
      r0 = pl.multiple_of(zbase + (i * K + b) * CHUNK, CHUNK)
      ds.append(pltpu.async_copy(agg_sh.at[pl.ds(r0, CHUNK)], rows_v.at[b],
                                 zsem))
    for d in ds:
      d.wait()
    ds = []
    for b in range(K):
      o0 = pl.multiple_of(obase + (i * K + b) * CHUNK, CHUNK)
      ds.append(pltpu.async_copy(rows_v.at[b], out_hbm.at[pl.ds(o0, CHUNK)],
                                 zsem))
    for d in ds:
      d.wait()
    return carry

  lax.fori_loop(0, ZCH // K, ogroup, 0)
  r0 = pl.multiple_of(zbase + (ZCH // K) * K * CHUNK, CHUNK)
  o0 = pl.multiple_of(obase + (ZCH // K) * K * CHUNK, CHUNK)
  pltpu.sync_copy(agg_sh.at[pl.ds(r0, CHUNK)], rows_v.at[0])
  pltpu.sync_copy(rows_v.at[0], out_hbm.at[pl.ds(o0, CHUNK)])


_sc_segsum = pl.kernel(
    _sc_body,
    out_type=jax.ShapeDtypeStruct((NC * N_ACC, D_H), jnp.float32),
    mesh=plsc.VectorSubcoreMesh(core_axis_name="c", subcore_axis_name="s"),
    scratch_types=[
        pltpu.VMEM((SG, CHUNK), jnp.int32),
        pltpu.VMEM((SG, CHUNK), jnp.int32),
        pltpu.VMEM((K, CHUNK, D_H), jnp.float32),
        pltpu.VMEM_SHARED((N_ACC, D_H), jnp.float32),
        pltpu.SemaphoreType.DMA,
        pltpu.SemaphoreType.DMA,
        pltpu.SemaphoreType.DMA,
    ],
    compiler_params=pltpu.CompilerParams(use_tc_tiling_on_sc=False),
)


def _prep_body(src_ref, dst_ref, et_ref, keys_ref, dstp_ref):
  keys = et_ref[...] * N + src_ref[...]
  pad_k = jnp.zeros((PB_OUT - PB_IN, CHUNK), jnp.int32)
  flat = jax.lax.broadcasted_iota(jnp.int32, (PB_OUT - PB_IN, CHUNK), 0) \
      * CHUNK + jax.lax.broadcasted_iota(
          jnp.int32, (PB_OUT - PB_IN, CHUNK), 1)
  pad_d = N + (flat % (N_ACC - N))
  keys_ref[...] = jnp.concatenate([keys, pad_k], axis=0)
  dstp_ref[...] = jnp.concatenate([dst_ref[...], pad_d], axis=0)


def _prep_edges(src2d, dst2d, et2d):
  return pl.pallas_call(
      _prep_body,
      grid=(PGRID,),
      in_specs=[
          pl.BlockSpec((PB_IN, CHUNK), lambda i: (i, 0)),
          pl.BlockSpec((PB_IN, CHUNK), lambda i: (i, 0)),
          pl.BlockSpec((PB_IN, CHUNK), lambda i: (i, 0)),
      ],
      out_specs=[
          pl.BlockSpec((PB_OUT, CHUNK), lambda i: (i, 0)),
          pl.BlockSpec((PB_OUT, CHUNK), lambda i: (i, 0)),
      ],
      out_shape=[
          jax.ShapeDtypeStruct((CH_TOTAL, CHUNK), jnp.int32),
          jax.ShapeDtypeStruct((CH_TOTAL, CHUNK), jnp.int32),
      ],
  )(src2d, dst2d, et2d)


def _transform1_body(x_ref, wr_ref, t_ref):
  xb = x_ref[...]
  for r in range(R):
    t_ref[r] = jnp.dot(xb, wr_ref[r], preferred_element_type=jnp.float32)


def _transform1(x, Wr1):
  return pl.pallas_call(
      _transform1_body,
      grid=(GRID_N,),
      in_specs=[
          pl.BlockSpec((BN, D_IN), lambda i: (i, 0)),
          pl.BlockSpec((R, D_IN, D_H), lambda i: (0, 0, 0)),
      ],
      out_specs=pl.BlockSpec((R, BN, D_H), lambda i: (0, i, 0)),
      out_shape=jax.ShapeDtypeStruct((R, N, D_H), jnp.float32),
  )(x, Wr1)


def _mid_body(a_ref, b_ref, x_ref, ws_ref, bias_ref, wr_ref, h_ref, t_ref):
  agg = a_ref[0] + b_ref[0]
  h = jnp.tanh(agg + jnp.dot(x_ref[...], ws_ref[...],
                             preferred_element_type=jnp.float32)
               + bias_ref[...])
  h_ref[...] = h
  for r in range(R):
    t_ref[r] = jnp.dot(h, wr_ref[r], preferred_element_type=jnp.float32)


def _layer_mid(aggp, x, Ws1, b1, Wr2):
  return pl.pallas_call(
      _mid_body,
      grid=(GRID_N,),
      in_specs=[
          pl.BlockSpec((1, BN, D_H), lambda i: (0, i, 0)),
          pl.BlockSpec((1, BN, D_H), lambda i: (1, i, 0)),
          pl.BlockSpec((BN, D_IN), lambda i: (i, 0)),
          pl.BlockSpec((D_IN, D_H), lambda i: (0, 0)),
          pl.BlockSpec((1, D_H), lambda i: (0, 0)),
          pl.BlockSpec((R, D_H, D_H), lambda i: (0, 0, 0)),
      ],
      out_specs=[
          pl.BlockSpec((BN, D_H), lambda i: (i, 0)),
          pl.BlockSpec((R, BN, D_H), lambda i: (0, i, 0)),
      ],
      out_shape=[
          jax.ShapeDtypeStruct((N, D_H), jnp.float32),
          jax.ShapeDtypeStruct((R, N, D_H), jnp.float32),
      ],
  )(aggp, aggp, x, Ws1, b1, Wr2)


def _head_body(a_ref, b_ref, h1_ref, x_ref, ws_ref, bias_ref,
               wi1a_ref, wi1b_ref, bi1_ref, wi2_ref, bi2_ref,
               wj1a_ref, wj1b_ref, bj1_ref, wj2_ref, bj2_ref,
               wf1_ref, bf1_ref, wf2_ref, bf2_ref,
               out_ref, acc_ref):
  i = pl.program_id(0)

  @pl.when(i == 0)
  def _():
    acc_ref[...] = jnp.zeros_like(acc_ref)

  agg = a_ref[0] + b_ref[0]
  h1 = h1_ref[...]
  xb = x_ref[...]
  h2 = jnp.tanh(agg + jnp.dot(h1, ws_ref[...],
                              preferred_element_type=jnp.float32)
                + bias_ref[...])
  u = jnp.maximum(
      jnp.dot(xb, wi1a_ref[...], preferred_element_type=jnp.float32)
      + jnp.dot(h2, wi1b_ref[...], preferred_element_type=jnp.float32)
      + bi1_ref[...], 0.0)
  io = jax.nn.sigmoid(jnp.dot(u, wi2_ref[...],
                              preferred_element_type=jnp.float32)
                      + bi2_ref[...])
  v = jnp.maximum(
      jnp.dot(xb, wj1a_ref[...], preferred_element_type=jnp.float32)
      + jnp.dot(h2, wj1b_ref[...], preferred_element_type=jnp.float32)
      + bj1_ref[...], 0.0)
  jo = jnp.tanh(jnp.dot(v, wj2_ref[...], preferred_element_type=jnp.float32)
                + bj2_ref[...])
  p = jnp.sum(io * jo, axis=0, keepdims=True)
  acc_ref[0:1, 0:D_H] = acc_ref[0:1, 0:D_H] + p

  @pl.when(i == GRID_N - 1)
  def _():
    g = jnp.tanh(acc_ref[0:1, 0:D_H])
    f = jnp.maximum(
        jnp.dot(g, wf1_ref[...], preferred_element_type=jnp.float32)
        + bf1_ref[...], 0.0)
    out_ref[...] = (jnp.dot(f, wf2_ref[...],
                            preferred_element_type=jnp.float32)
                    + bf2_ref[...])


def _head(aggp, h1, x, Ws2, b2, Wi1a, Wi1b, bi1, Wi2, bi2,
          Wj1a, Wj1b, bj1, Wj2, bj2, Wf1, bf1, Wf2, bf2):
  full = lambda shape: pl.BlockSpec(shape, lambda i: tuple(0 for _ in shape))
  return pl.pallas_call(
      _head_body,
      grid=(GRID_N,),
      in_specs=[
          pl.BlockSpec((1, BN, D_H), lambda i: (0, i, 0)),
          pl.BlockSpec((1, BN, D_H), lambda i: (1, i, 0)),
          pl.BlockSpec((BN, D_H), lambda i: (i, 0)),
          pl.BlockSpec((BN, D_IN), lambda i: (i, 0)),
          full((D_H, D_H)), full((1, D_H)),
          full((D_IN, 64)), full((D_H, 64)), full((1, 64)),
          full((64, D_H)), full((1, D_H)),
          full((D_IN, 64)), full((D_H, 64)), full((1, 64)),
          full((64, D_H)), full((1, D_H)),
          full((D_H, 64)), full((1, 64)), full((64, 1)), full((1, 1)),
      ],
      out_specs=pl.BlockSpec((1, 1), lambda i: (0, 0)),
      out_shape=jax.ShapeDtypeStruct((1, 1), jnp.float32),
      scratch_shapes=[pltpu.VMEM((8, 128), jnp.float32)],
  )(aggp, aggp, h1, x, Ws2, b2, Wi1a, Wi1b, bi1, Wi2, bi2,
    Wj1a, Wj1b, bj1, Wj2, bj2, Wf1, bf1, Wf2, bf2)


def kernel(x, edge_index, edge_type, Wr1, Ws1, b1, Wr2, Ws2, b2,
           Wi1, bi1, Wi2, bi2, Wj1, bj1, Wj2, bj2, Wf1, bf1, Wf2, bf2):
  src2d = edge_index[0].reshape(EROWS_IN, CHUNK)
  dst2d = edge_index[1].reshape(EROWS_IN, CHUNK)
  et2d = edge_type.reshape(EROWS_IN, CHUNK)

  keys2d, dstp2d = _prep_edges(src2d, dst2d, et2d)
  zeros128 = jnp.zeros((CHUNK, D_H), jnp.float32)

  t1 = _transform1(x, Wr1)
  aggp1 = _sc_segsum(keys2d, dstp2d, t1.reshape(R * N, D_H), zeros128)
  h1, t2 = _layer_mid(aggp1.reshape(NC, N_ACC, D_H), x, Ws1,
                      b1.reshape(1, D_H), Wr2)
  aggp2 = _sc_segsum(keys2d, dstp2d, t2.reshape(R * N, D_H), zeros128)
  out = _head(aggp2.reshape(NC, N_ACC, D_H), h1, x, Ws2,
              b2.reshape(1, D_H),
              Wi1[:D_IN], Wi1[D_IN:], bi1.reshape(1, 64),
              Wi2, bi2.reshape(1, D_H),
              Wj1[:D_IN], Wj1[D_IN:], bj1.reshape(1, 64),
              Wj2, bj2.reshape(1, D_H),
              Wf1, bf1.reshape(1, 64), Wf2, bf2.reshape(1, 1))
  return out.reshape(1)


# speedup vs baseline: 17.7803x; 1.0587x over previous
import jax
import jax.numpy as jnp
from jax import lax
from jax.experimental import pallas as pl
from jax.experimental.pallas import tpu as pltpu
from jax.experimental.pallas import tpu_sc as plsc

N = 50000
E = 800000
R = 4
D_IN = 16
D_H = 32

NC = 2
NS = 16
NW = NC * NS
CHUNK = 128
E_PAD = 819200
CH_TOTAL = E_PAD // CHUNK
CH_PER_TILE = CH_TOTAL // NW
K = 5
SG = 20
NSG0 = 13
NSG1 = 7
N_ACC = 51200
ZROWS = N_ACC // NS
ZCH = ZROWS // CHUNK

EROWS_IN = E // CHUNK
PB_IN = EROWS_IN
PB_OUT = CH_TOTAL
PGRID = 1

BN = 2000
GRID_N = N // BN


def _sc_body(keys_hbm, dst_hbm, table_hbm, zeros_hbm, out_hbm,
             keysb, dstb, rows_v, agg_sh, gsem, ssem, zsem):
  c = lax.axis_index("c")
  s = lax.axis_index("s")
  tid = c * NS + s
  zbase = s * ZROWS

  pltpu.sync_copy(zeros_hbm, rows_v.at[0])

  def zgroup(i, carry):
    ds = []
    for b in range(K):
      r0 = pl.multiple_of(zbase + (i * K + b) * CHUNK, CHUNK)
      ds.append(pltpu.async_copy(rows_v.at[0], agg_sh.at[pl.ds(r0, CHUNK)],
                                 zsem))
    for d in ds:
      d.wait()
    return carry

  lax.fori_loop(0, ZCH // K, zgroup, 0)
  r0 = pl.multiple_of(zbase + (ZCH // K) * K * CHUNK, CHUNK)
  pltpu.sync_copy(rows_v.at[0], agg_sh.at[pl.ds(r0, CHUNK)])

  nsg = jnp.where(c == 0, NSG0, NSG1)
  cbase = pl.multiple_of(
      jnp.where(c == 0, s * (NSG0 * SG),
                16 * NSG0 * SG + s * (NSG1 * SG)), 8)
  plsc.subcore_barrier()

  def sg_body(sg, carry):
    i0 = pl.multiple_of(cbase + sg * SG, 8)
    pltpu.sync_copy(keys_hbm.at[pl.ds(i0, SG)], keysb)
    pltpu.sync_copy(dst_hbm.at[pl.ds(i0, SG)], dstb)

    def group_body(g, carry2):
      gds = [pltpu.async_copy(table_hbm.at[keysb.at[g * K + b]],
                              rows_v.at[b], gsem) for b in range(K)]
      for d in gds:
        d.wait()
      sds = [pltpu.async_copy(rows_v.at[b], agg_sh.at[dstb.at[g * K + b]],
                              ssem, add=True) for b in range(K)]
      for d in sds:
        d.wait()
      return carry2

    lax.fori_loop(0, SG // K, group_body, 0)
    return carry

  lax.fori_loop(0, nsg, sg_body, 0)
  plsc.subcore_barrier()

  obase = c * N_ACC + s * ZROWS

  def ogroup(i, carry):
    ds = []
    for b in range(K):
      r0 = pl.multiple_of(zbase + (i * K + b) * CHUNK, CHUNK)
      ds.append(pltpu.async_copy(agg_sh.at[pl.ds(r0, CHUNK)], rows_v.at[b],
                                 zsem))
    for d in ds:
      d.wait()
    ds = []
    for b in range(K):
      o0 = pl.multiple_of(obase + (i * K + b) * CHUNK, CHUNK)
      ds.append(pltpu.async_copy(rows_v.at[b], out_hbm.at[pl.ds(o0, CHUNK)],
                                 zsem))
    for d in ds:
      d.wait()
    return carry

  lax.fori_loop(0, ZCH // K, ogroup, 0)
  r0 = pl.multiple_of(zbase + (ZCH // K) * K * CHUNK, CHUNK)
  o0 = pl.multiple_of(obase + (ZCH // K) * K * CHUNK, CHUNK)
  pltpu.sync_copy(agg_sh.at[pl.ds(r0, CHUNK)], rows_v.at[0])
  pltpu.sync_copy(rows_v.at[0], out_hbm.at[pl.ds(o0, CHUNK)])


_sc_segsum = pl.kernel(
    _sc_body,
    out_type=jax.ShapeDtypeStruct((NC * N_ACC, D_H), jnp.float32),
    mesh=plsc.VectorSubcoreMesh(core_axis_name="c", subcore_axis_name="s"),
    scratch_types=[
        pltpu.VMEM((SG, CHUNK), jnp.int32),
        pltpu.VMEM((SG, CHUNK), jnp.int32),
        pltpu.VMEM((K, CHUNK, D_H), jnp.float32),
        pltpu.VMEM_SHARED((N_ACC, D_H), jnp.float32),
        pltpu.SemaphoreType.DMA,
        pltpu.SemaphoreType.DMA,
        pltpu.SemaphoreType.DMA,
    ],
    compiler_params=pltpu.CompilerParams(use_tc_tiling_on_sc=False),
)


def _prep_body(src_ref, dst_ref, et_ref, keys_ref, dstp_ref):
  keys = et_ref[...] * N + src_ref[...]
  pad_k = jnp.zeros((PB_OUT - PB_IN, CHUNK), jnp.int32)
  flat = jax.lax.broadcasted_iota(jnp.int32, (PB_OUT - PB_IN, CHUNK), 0) \
      * CHUNK + jax.lax.broadcasted_iota(
          jnp.int32, (PB_OUT - PB_IN, CHUNK), 1)
  pad_d = N + (flat % (N_ACC - N))
  keys_ref[...] = jnp.concatenate([keys, pad_k], axis=0)
  dstp_ref[...] = jnp.concatenate([dst_ref[...], pad_d], axis=0)


def _prep_edges(src2d, dst2d, et2d):
  return pl.pallas_call(
      _prep_body,
      grid=(PGRID,),
      in_specs=[
          pl.BlockSpec((PB_IN, CHUNK), lambda i: (i, 0)),
          pl.BlockSpec((PB_IN, CHUNK), lambda i: (i, 0)),
          pl.BlockSpec((PB_IN, CHUNK), lambda i: (i, 0)),
      ],
      out_specs=[
          pl.BlockSpec((PB_OUT, CHUNK), lambda i: (i, 0)),
          pl.BlockSpec((PB_OUT, CHUNK), lambda i: (i, 0)),
      ],
      out_shape=[
          jax.ShapeDtypeStruct((CH_TOTAL, CHUNK), jnp.int32),
          jax.ShapeDtypeStruct((CH_TOTAL, CHUNK), jnp.int32),
      ],
  )(src2d, dst2d, et2d)


def _transform1_body(x_ref, wr_ref, t_ref):
  xb = x_ref[...]
  for r in range(R):
    t_ref[r] = jnp.dot(xb, wr_ref[r], preferred_element_type=jnp.float32)


def _transform1(x, Wr1):
  return pl.pallas_call(
      _transform1_body,
      grid=(GRID_N,),
      in_specs=[
          pl.BlockSpec((BN, D_IN), lambda i: (i, 0)),
          pl.BlockSpec((R, D_IN, D_H), lambda i: (0, 0, 0)),
      ],
      out_specs=pl.BlockSpec((R, BN, D_H), lambda i: (0, i, 0)),
      out_shape=jax.ShapeDtypeStruct((R, N, D_H), jnp.float32),
  )(x, Wr1)


def _mid_body(a_ref, b_ref, x_ref, ws_ref, bias_ref, wr_ref, h_ref, t_ref):
  agg = a_ref[0] + b_ref[0]
  h = jnp.tanh(agg + jnp.dot(x_ref[...], ws_ref[...],
                             preferred_element_type=jnp.float32)
               + bias_ref[...])
  h_ref[...] = h
  for r in range(R):
    t_ref[r] = jnp.dot(h, wr_ref[r], preferred_element_type=jnp.float32)


def _layer_mid(aggp, x, Ws1, b1, Wr2):
  return pl.pallas_call(
      _mid_body,
      grid=(GRID_N,),
      in_specs=[
          pl.BlockSpec((1, BN, D_H), lambda i: (0, i, 0)),
          pl.BlockSpec((1, BN, D_H), lambda i: (1, i, 0)),
          pl.BlockSpec((BN, D_IN), lambda i: (i, 0)),
          pl.BlockSpec((D_IN, D_H), lambda i: (0, 0)),
          pl.BlockSpec((1, D_H), lambda i: (0, 0)),
          pl.BlockSpec((R, D_H, D_H), lambda i: (0, 0, 0)),
      ],
      out_specs=[
          pl.BlockSpec((BN, D_H), lambda i: (i, 0)),
          pl.BlockSpec((R, BN, D_H), lambda i: (0, i, 0)),
      ],
      out_shape=[
          jax.ShapeDtypeStruct((N, D_H), jnp.float32),
          jax.ShapeDtypeStruct((R, N, D_H), jnp.float32),
      ],
  )(aggp, aggp, x, Ws1, b1, Wr2)


def _head_body(a_ref, b_ref, h1_ref, x_ref, ws_ref, bias_ref,
               wi1a_ref, wi1b_ref, bi1_ref, wi2_ref, bi2_ref,
               wj1a_ref, wj1b_ref, bj1_ref, wj2_ref, bj2_ref,
               wf1_ref, bf1_ref, wf2_ref, bf2_ref,
               out_ref, acc_ref):
  i = pl.program_id(0)

  @pl.when(i == 0)
  def _():
    acc_ref[...] = jnp.zeros_like(acc_ref)

  agg = a_ref[0] + b_ref[0]
  h1 = h1_ref[...]
  xb = x_ref[...]
  h2 = jnp.tanh(agg + jnp.dot(h1, ws_ref[...],
                              preferred_element_type=jnp.float32)
                + bias_ref[...])
  u = jnp.maximum(
      jnp.dot(xb, wi1a_ref[...], preferred_element_type=jnp.float32)
      + jnp.dot(h2, wi1b_ref[...], preferred_element_type=jnp.float32)
      + bi1_ref[...], 0.0)
  io = jax.nn.sigmoid(jnp.dot(u, wi2_ref[...],
                              preferred_element_type=jnp.float32)
                      + bi2_ref[...])
  v = jnp.maximum(
      jnp.dot(xb, wj1a_ref[...], preferred_element_type=jnp.float32)
      + jnp.dot(h2, wj1b_ref[...], preferred_element_type=jnp.float32)
      + bj1_ref[...], 0.0)
  jo = jnp.tanh(jnp.dot(v, wj2_ref[...], preferred_element_type=jnp.float32)
                + bj2_ref[...])
  p = jnp.sum(io * jo, axis=0, keepdims=True)
  acc_ref[0:1, 0:D_H] = acc_ref[0:1, 0:D_H] + p

  @pl.when(i == GRID_N - 1)
  def _():
    g = jnp.tanh(acc_ref[0:1, 0:D_H])
    f = jnp.maximum(
        jnp.dot(g, wf1_ref[...], preferred_element_type=jnp.float32)
        + bf1_ref[...], 0.0)
    out_ref[...] = (jnp.dot(f, wf2_ref[...],
                            preferred_element_type=jnp.float32)
                    + bf2_ref[...])


def _head(aggp, h1, x, Ws2, b2, Wi1a, Wi1b, bi1, Wi2, bi2,
          Wj1a, Wj1b, bj1, Wj2, bj2, Wf1, bf1, Wf2, bf2):
  full = lambda shape: pl.BlockSpec(shape, lambda i: tuple(0 for _ in shape))
  return pl.pallas_call(
      _head_body,
      grid=(GRID_N,),
      in_specs=[
          pl.BlockSpec((1, BN, D_H), lambda i: (0, i, 0)),
          pl.BlockSpec((1, BN, D_H), lambda i: (1, i, 0)),
          pl.BlockSpec((BN, D_H), lambda i: (i, 0)),
          pl.BlockSpec((BN, D_IN), lambda i: (i, 0)),
          full((D_H, D_H)), full((1, D_H)),
          full((D_IN, 64)), full((D_H, 64)), full((1, 64)),
          full((64, D_H)), full((1, D_H)),
          full((D_IN, 64)), full((D_H, 64)), full((1, 64)),
          full((64, D_H)), full((1, D_H)),
          full((D_H, 64)), full((1, 64)), full((64, 1)), full((1, 1)),
      ],
      out_specs=pl.BlockSpec((1, 1), lambda i: (0, 0)),
      out_shape=jax.ShapeDtypeStruct((1, 1), jnp.float32),
      scratch_shapes=[pltpu.VMEM((8, 128), jnp.float32)],
  )(aggp, aggp, h1, x, Ws2, b2, Wi1a, Wi1b, bi1, Wi2, bi2,
    Wj1a, Wj1b, bj1, Wj2, bj2, Wf1, bf1, Wf2, bf2)


def kernel(x, edge_index, edge_type, Wr1, Ws1, b1, Wr2, Ws2, b2,
           Wi1, bi1, Wi2, bi2, Wj1, bj1, Wj2, bj2, Wf1, bf1, Wf2, bf2):
  src2d = edge_index[0].reshape(EROWS_IN, CHUNK)
  dst2d = edge_index[1].reshape(EROWS_IN, CHUNK)
  et2d = edge_type.reshape(EROWS_IN, CHUNK)

  keys2d, dstp2d = _prep_edges(src2d, dst2d, et2d)
  zeros128 = jnp.zeros((CHUNK, D_H), jnp.float32)

  t1 = _transform1(x, Wr1)
  aggp1 = _sc_segsum(keys2d, dstp2d, t1.reshape(R * N, D_H), zeros128)
  h1, t2 = _layer_mid(aggp1.reshape(NC, N_ACC, D_H), x, Ws1,
                      b1.reshape(1, D_H), Wr2)
  aggp2 = _sc_segsum(keys2d, dstp2d, t2.reshape(R * N, D_H), zeros128)
  out = _head(aggp2.reshape(NC, N_ACC, D_H), h1, x, Ws2,
              b2.reshape(1, D_H),
              Wi1[:D_IN], Wi1[D_IN:], bi1.reshape(1, 64),
              Wi2, bi2.reshape(1, D_H),
              Wj1[:D_IN], Wj1[D_IN:], bj1.reshape(1, 64),
              Wj2, bj2.reshape(1, D_H),
              Wf1, bf1.reshape(1, 64), Wf2, bf2.reshape(1, 1))
  return out.reshape(1)


# speedup vs baseline: 18.6668x; 1.0499x over previous
import jax
import jax.numpy as jnp
from jax import lax
from jax.experimental import pallas as pl
from jax.experimental.pallas import tpu as pltpu
from jax.experimental.pallas import tpu_sc as plsc

N = 50000
E = 800000
R = 4
D_IN = 16
D_H = 32

NC = 2
NS = 16
NW = NC * NS
CHUNK = 128
E_PAD = 819200
CH_TOTAL = E_PAD // CHUNK
CH_PER_TILE = CH_TOTAL // NW
K = 5
SG = 20
NSG0 = 15
NSG1 = 5
N_ACC = 51200
ZROWS = N_ACC // NS
ZCH = ZROWS // CHUNK

EROWS_IN = E // CHUNK
PB_IN = EROWS_IN
PB_OUT = CH_TOTAL
PGRID = 1

BN = 2000
GRID_N = N // BN


def _sc_body(keys_hbm, dst_hbm, table_hbm, zeros_hbm, out_hbm,
             keysb, dstb, rows_v, agg_sh, gsem, ssem, zsem):
  c = lax.axis_index("c")
  s = lax.axis_index("s")
  tid = c * NS + s
  zbase = s * ZROWS

  pltpu.sync_copy(zeros_hbm, rows_v.at[0])

  def zgroup(i, carry):
    ds = []
    for b in range(K):
      r0 = pl.multiple_of(zbase + (i * K + b) * CHUNK, CHUNK)
      ds.append(pltpu.async_copy(rows_v.at[0], agg_sh.at[pl.ds(r0, CHUNK)],
                                 zsem))
    for d in ds:
      d.wait()
    return carry

  lax.fori_loop(0, ZCH // K, zgroup, 0)
  r0 = pl.multiple_of(zbase + (ZCH // K) * K * CHUNK, CHUNK)
  pltpu.sync_copy(rows_v.at[0], agg_sh.at[pl.ds(r0, CHUNK)])

  nsg = jnp.where(c == 0, NSG0, NSG1)
  cbase = pl.multiple_of(
      jnp.where(c == 0, s * (NSG0 * SG),
                16 * NSG0 * SG + s * (NSG1 * SG)), 8)
  plsc.subcore_barrier()

  def sg_body(sg, carry):
    i0 = pl.multiple_of(cbase + sg * SG, 8)
    pltpu.sync_copy(keys_hbm.at[pl.ds(i0, SG)], keysb)
    pltpu.sync_copy(dst_hbm.at[pl.ds(i0, SG)], dstb)

    def group_body(g, carry2):
      gds = [pltpu.async_copy(table_hbm.at[keysb.at[g * K + b]],
                              rows_v.at[b], gsem) for b in range(K)]
      for d in gds:
        d.wait()
      sds = [pltpu.async_copy(rows_v.at[b], agg_sh.at[dstb.at[g * K + b]],
                              ssem, add=True) for b in range(K)]
      for d in sds:
        d.wait()
      return carry2

    lax.fori_loop(0, SG // K, group_body, 0)
    return carry

  lax.fori_loop(0, nsg, sg_body, 0)
  plsc.subcore_barrier()

  obase = c * N_ACC + s * ZROWS

  def ogroup(i, carry):
    ds = []
    for b in range(K):
      r0 = pl.multiple_of(zbase + (i * K + b) * CHUNK, CHUNK)
      ds.append(pltpu.async_copy(agg_sh.at[pl.ds(r0, CHUNK)], rows_v.at[b],
                                 zsem))
    for d in ds:
      d.wait()
    ds = []
    for b in range(K):
      o0 = pl.multiple_of(obase + (i * K + b) * CHUNK, CHUNK)
      ds.append(pltpu.async_copy(rows_v.at[b], out_hbm.at[pl.ds(o0, CHUNK)],
                                 zsem))
    for d in ds:
      d.wait()
    return carry

  lax.fori_loop(0, ZCH // K, ogroup, 0)
  r0 = pl.multiple_of(zbase + (ZCH // K) * K * CHUNK, CHUNK)
  o0 = pl.multiple_of(obase + (ZCH // K) * K * CHUNK, CHUNK)
  pltpu.sync_copy(agg_sh.at[pl.ds(r0, CHUNK)], rows_v.at[0])
  pltpu.sync_copy(rows_v.at[0], out_hbm.at[pl.ds(o0, CHUNK)])


_sc_segsum = pl.kernel(
    _sc_body,
    out_type=jax.ShapeDtypeStruct((NC * N_ACC, D_H), jnp.float32),
    mesh=plsc.VectorSubcoreMesh(core_axis_name="c", subcore_axis_name="s"),
    scratch_types=[
        pltpu.VMEM((SG, CHUNK), jnp.int32),
        pltpu.VMEM((SG, CHUNK), jnp.int32),
        pltpu.VMEM((K, CHUNK, D_H), jnp.float32),
        pltpu.VMEM_SHARED((N_ACC, D_H), jnp.float32),
        pltpu.SemaphoreType.DMA,
        pltpu.SemaphoreType.DMA,
        pltpu.SemaphoreType.DMA,
    ],
    compiler_params=pltpu.CompilerParams(use_tc_tiling_on_sc=False),
)


def _prep_body(src_ref, dst_ref, et_ref, keys_ref, dstp_ref):
  keys = et_ref[...] * N + src_ref[...]
  pad_k = jnp.zeros((PB_OUT - PB_IN, CHUNK), jnp.int32)
  flat = jax.lax.broadcasted_iota(jnp.int32, (PB_OUT - PB_IN, CHUNK), 0) \
      * CHUNK + jax.lax.broadcasted_iota(
          jnp.int32, (PB_OUT - PB_IN, CHUNK), 1)
  pad_d = N + (flat % (N_ACC - N))
  keys_ref[...] = jnp.concatenate([keys, pad_k], axis=0)
  dstp_ref[...] = jnp.concatenate([dst_ref[...], pad_d], axis=0)


def _prep_edges(src2d, dst2d, et2d):
  return pl.pallas_call(
      _prep_body,
      grid=(PGRID,),
      in_specs=[
          pl.BlockSpec((PB_IN, CHUNK), lambda i: (i, 0)),
          pl.BlockSpec((PB_IN, CHUNK), lambda i: (i, 0)),
          pl.BlockSpec((PB_IN, CHUNK), lambda i: (i, 0)),
      ],
      out_specs=[
          pl.BlockSpec((PB_OUT, CHUNK), lambda i: (i, 0)),
          pl.BlockSpec((PB_OUT, CHUNK), lambda i: (i, 0)),
      ],
      out_shape=[
          jax.ShapeDtypeStruct((CH_TOTAL, CHUNK), jnp.int32),
          jax.ShapeDtypeStruct((CH_TOTAL, CHUNK), jnp.int32),
      ],
  )(src2d, dst2d, et2d)


def _transform1_body(x_ref, wr_ref, t_ref):
  xb = x_ref[...]
  for r in range(R):
    t_ref[r] = jnp.dot(xb, wr_ref[r], preferred_element_type=jnp.float32)


def _transform1(x, Wr1):
  return pl.pallas_call(
      _transform1_body,
      grid=(GRID_N,),
      in_specs=[
          pl.BlockSpec((BN, D_IN), lambda i: (i, 0)),
          pl.BlockSpec((R, D_IN, D_H), lambda i: (0, 0, 0)),
      ],
      out_specs=pl.BlockSpec((R, BN, D_H), lambda i: (0, i, 0)),
      out_shape=jax.ShapeDtypeStruct((R, N, D_H), jnp.float32),
  )(x, Wr1)


def _mid_body(a_ref, b_ref, x_ref, ws_ref, bias_ref, wr_ref, h_ref, t_ref):
  agg = a_ref[0] + b_ref[0]
  h = jnp.tanh(agg + jnp.dot(x_ref[...], ws_ref[...],
                             preferred_element_type=jnp.float32)
               + bias_ref[...])
  h_ref[...] = h
  for r in range(R):
    t_ref[r] = jnp.dot(h, wr_ref[r], preferred_element_type=jnp.float32)


def _layer_mid(aggp, x, Ws1, b1, Wr2):
  return pl.pallas_call(
      _mid_body,
      grid=(GRID_N,),
      in_specs=[
          pl.BlockSpec((1, BN, D_H), lambda i: (0, i, 0)),
          pl.BlockSpec((1, BN, D_H), lambda i: (1, i, 0)),
          pl.BlockSpec((BN, D_IN), lambda i: (i, 0)),
          pl.BlockSpec((D_IN, D_H), lambda i: (0, 0)),
          pl.BlockSpec((1, D_H), lambda i: (0, 0)),
          pl.BlockSpec((R, D_H, D_H), lambda i: (0, 0, 0)),
      ],
      out_specs=[
          pl.BlockSpec((BN, D_H), lambda i: (i, 0)),
          pl.BlockSpec((R, BN, D_H), lambda i: (0, i, 0)),
      ],
      out_shape=[
          jax.ShapeDtypeStruct((N, D_H), jnp.float32),
          jax.ShapeDtypeStruct((R, N, D_H), jnp.float32),
      ],
  )(aggp, aggp, x, Ws1, b1, Wr2)


def _head_body(a_ref, b_ref, h1_ref, x_ref, ws_ref, bias_ref,
               wi1a_ref, wi1b_ref, bi1_ref, wi2_ref, bi2_ref,
               wj1a_ref, wj1b_ref, bj1_ref, wj2_ref, bj2_ref,
               wf1_ref, bf1_ref, wf2_ref, bf2_ref,
               out_ref, acc_ref):
  i = pl.program_id(0)

  @pl.when(i == 0)
  def _():
    acc_ref[...] = jnp.zeros_like(acc_ref)

  agg = a_ref[0] + b_ref[0]
  h1 = h1_ref[...]
  xb = x_ref[...]
  h2 = jnp.tanh(agg + jnp.dot(h1, ws_ref[...],
                              preferred_element_type=jnp.float32)
                + bias_ref[...])
  u = jnp.maximum(
      jnp.dot(xb, wi1a_ref[...], preferred_element_type=jnp.float32)
      + jnp.dot(h2, wi1b_ref[...], preferred_element_type=jnp.float32)
      + bi1_ref[...], 0.0)
  io = jax.nn.sigmoid(jnp.dot(u, wi2_ref[...],
                              preferred_element_type=jnp.float32)
                      + bi2_ref[...])
  v = jnp.maximum(
      jnp.dot(xb, wj1a_ref[...], preferred_element_type=jnp.float32)
      + jnp.dot(h2, wj1b_ref[...], preferred_element_type=jnp.float32)
      + bj1_ref[...], 0.0)
  jo = jnp.tanh(jnp.dot(v, wj2_ref[...], preferred_element_type=jnp.float32)
                + bj2_ref[...])
  p = jnp.sum(io * jo, axis=0, keepdims=True)
  acc_ref[0:1, 0:D_H] = acc_ref[0:1, 0:D_H] + p

  @pl.when(i == GRID_N - 1)
  def _():
    g = jnp.tanh(acc_ref[0:1, 0:D_H])
    f = jnp.maximum(
        jnp.dot(g, wf1_ref[...], preferred_element_type=jnp.float32)
        + bf1_ref[...], 0.0)
    out_ref[...] = (jnp.dot(f, wf2_ref[...],
                            preferred_element_type=jnp.float32)
                    + bf2_ref[...])


def _head(aggp, h1, x, Ws2, b2, Wi1a, Wi1b, bi1, Wi2, bi2,
          Wj1a, Wj1b, bj1, Wj2, bj2, Wf1, bf1, Wf2, bf2):
  full = lambda shape: pl.BlockSpec(shape, lambda i: tuple(0 for _ in shape))
  return pl.pallas_call(
      _head_body,
      grid=(GRID_N,),
      in_specs=[
          pl.BlockSpec((1, BN, D_H), lambda i: (0, i, 0)),
          pl.BlockSpec((1, BN, D_H), lambda i: (1, i, 0)),
          pl.BlockSpec((BN, D_H), lambda i: (i, 0)),
          pl.BlockSpec((BN, D_IN), lambda i: (i, 0)),
          full((D_H, D_H)), full((1, D_H)),
          full((D_IN, 64)), full((D_H, 64)), full((1, 64)),
          full((64, D_H)), full((1, D_H)),
          full((D_IN, 64)), full((D_H, 64)), full((1, 64)),
          full((64, D_H)), full((1, D_H)),
          full((D_H, 64)), full((1, 64)), full((64, 1)), full((1, 1)),
      ],
      out_specs=pl.BlockSpec((1, 1), lambda i: (0, 0)),
      out_shape=jax.ShapeDtypeStruct((1, 1), jnp.float32),
      scratch_shapes=[pltpu.VMEM((8, 128), jnp.float32)],
  )(aggp, aggp, h1, x, Ws2, b2, Wi1a, Wi1b, bi1, Wi2, bi2,
    Wj1a, Wj1b, bj1, Wj2, bj2, Wf1, bf1, Wf2, bf2)


def kernel(x, edge_index, edge_type, Wr1, Ws1, b1, Wr2, Ws2, b2,
           Wi1, bi1, Wi2, bi2, Wj1, bj1, Wj2, bj2, Wf1, bf1, Wf2, bf2):
  src2d = edge_index[0].reshape(EROWS_IN, CHUNK)
  dst2d = edge_index[1].reshape(EROWS_IN, CHUNK)
  et2d = edge_type.reshape(EROWS_IN, CHUNK)

  keys2d, dstp2d = _prep_edges(src2d, dst2d, et2d)
  zeros128 = jnp.zeros((CHUNK, D_H), jnp.float32)

  t1 = _transform1(x, Wr1)
  aggp1 = _sc_segsum(keys2d, dstp2d, t1.reshape(R * N, D_H), zeros128)
  h1, t2 = _layer_mid(aggp1.reshape(NC, N_ACC, D_H), x, Ws1,
                      b1.reshape(1, D_H), Wr2)
  aggp2 = _sc_segsum(keys2d, dstp2d, t2.reshape(R * N, D_H), zeros128)
  out = _head(aggp2.reshape(NC, N_ACC, D_H), h1, x, Ws2,
              b2.reshape(1, D_H),
              Wi1[:D_IN], Wi1[D_IN:], bi1.reshape(1, 64),
              Wi2, bi2.reshape(1, D_H),
              Wj1[:D_IN], Wj1[D_IN:], bj1.reshape(1, 64),
              Wj2, bj2.reshape(1, D_H),
              Wf1, bf1.reshape(1, 64), Wf2, bf2.reshape(1, 1))
  return out.reshape(1)


# speedup vs baseline: 19.0596x; 1.0210x over previous
import jax
import jax.numpy as jnp
from jax import lax
from jax.experimental import pallas as pl
from jax.experimental.pallas import tpu as pltpu
from jax.experimental.pallas import tpu_sc as plsc

N = 50000
E = 800000
R = 4
D_IN = 16
D_H = 32

NC = 2
NS = 16
NW = NC * NS
CHUNK = 128
E_PAD = 819200
CH_TOTAL = E_PAD // CHUNK
CH_PER_TILE = CH_TOTAL // NW
K = 5
SG = 20
NSG0 = 16
NSG1 = 4
N_ACC = 51200
ZROWS = N_ACC // NS
ZCH = ZROWS // CHUNK

EROWS_IN = E // CHUNK
PB_IN = EROWS_IN
PB_OUT = CH_TOTAL
PGRID = 1

BN = 2000
GRID_N = N // BN


def _sc_body(keys_hbm, dst_hbm, table_hbm, zeros_hbm, out_hbm,
             keysb, dstb, rows_v, agg_sh, gsem, ssem, zsem):
  c = lax.axis_index("c")
  s = lax.axis_index("s")
  tid = c * NS + s
  zbase = s * ZROWS

  pltpu.sync_copy(zeros_hbm, rows_v.at[0])

  def zgroup(i, carry):
    ds = []
    for b in range(K):
      r0 = pl.multiple_of(zbase + (i * K + b) * CHUNK, CHUNK)
      ds.append(pltpu.async_copy(rows_v.at[0], agg_sh.at[pl.ds(r0, CHUNK)],
                                 zsem))
    for d in ds:
      d.wait()
    return carry

  lax.fori_loop(0, ZCH // K, zgroup, 0)
  r0 = pl.multiple_of(zbase + (ZCH // K) * K * CHUNK, CHUNK)
  pltpu.sync_copy(rows_v.at[0], agg_sh.at[pl.ds(r0, CHUNK)])

  nsg = jnp.where(c == 0, NSG0, NSG1)
  cbase = pl.multiple_of(
      jnp.where(c == 0, s * (NSG0 * SG),
                16 * NSG0 * SG + s * (NSG1 * SG)), 8)
  plsc.subcore_barrier()

  def sg_body(sg, carry):
    i0 = pl.multiple_of(cbase + sg * SG, 8)
    pltpu.sync_copy(keys_hbm.at[pl.ds(i0, SG)], keysb)
    pltpu.sync_copy(dst_hbm.at[pl.ds(i0, SG)], dstb)

    def group_body(g, carry2):
      gds = [pltpu.async_copy(table_hbm.at[keysb.at[g * K + b]],
                              rows_v.at[b], gsem) for b in range(K)]
      for d in gds:
        d.wait()
      sds = [pltpu.async_copy(rows_v.at[b], agg_sh.at[dstb.at[g * K + b]],
                              ssem, add=True) for b in range(K)]
      for d in sds:
        d.wait()
      return carry2

    lax.fori_loop(0, SG // K, group_body, 0)
    return carry

  lax.fori_loop(0, nsg, sg_body, 0)
  plsc.subcore_barrier()

  obase = c * N_ACC + s * ZROWS

  def ogroup(i, carry):
    ds = []
    for b in range(K):
      r0 = pl.multiple_of(zbase + (i * K + b) * CHUNK, CHUNK)
      ds.append(pltpu.async_copy(agg_sh.at[pl.ds(r0, CHUNK)], rows_v.at[b],
                                 zsem))
    for d in ds:
      d.wait()
    ds = []
    for b in range(K):
      o0 = pl.multiple_of(obase + (i * K + b) * CHUNK, CHUNK)
      ds.append(pltpu.async_copy(rows_v.at[b], out_hbm.at[pl.ds(o0, CHUNK)],
                                 zsem))
    for d in ds:
      d.wait()
    return carry

  lax.fori_loop(0, ZCH // K, ogroup, 0)
  r0 = pl.multiple_of(zbase + (ZCH // K) * K * CHUNK, CHUNK)
  o0 = pl.multiple_of(obase + (ZCH // K) * K * CHUNK, CHUNK)
  pltpu.sync_copy(agg_sh.at[pl.ds(r0, CHUNK)], rows_v.at[0])
  pltpu.sync_copy(rows_v.at[0], out_hbm.at[pl.ds(o0, CHUNK)])


_sc_segsum = pl.kernel(
    _sc_body,
    out_type=jax.ShapeDtypeStruct((NC * N_ACC, D_H), jnp.float32),
    mesh=plsc.VectorSubcoreMesh(core_axis_name="c", subcore_axis_name="s"),
    scratch_types=[
        pltpu.VMEM((SG, CHUNK), jnp.int32),
        pltpu.VMEM((SG, CHUNK), jnp.int32),
        pltpu.VMEM((K, CHUNK, D_H), jnp.float32),
        pltpu.VMEM_SHARED((N_ACC, D_H), jnp.float32),
        pltpu.SemaphoreType.DMA,
        pltpu.SemaphoreType.DMA,
        pltpu.SemaphoreType.DMA,
    ],
    compiler_params=pltpu.CompilerParams(use_tc_tiling_on_sc=False),
)


def _prep_body(src_ref, dst_ref, et_ref, keys_ref, dstp_ref):
  keys = et_ref[...] * N + src_ref[...]
  pad_k = jnp.zeros((PB_OUT - PB_IN, CHUNK), jnp.int32)
  flat = jax.lax.broadcasted_iota(jnp.int32, (PB_OUT - PB_IN, CHUNK), 0) \
      * CHUNK + jax.lax.broadcasted_iota(
          jnp.int32, (PB_OUT - PB_IN, CHUNK), 1)
  pad_d = N + (flat % (N_ACC - N))
  keys_ref[...] = jnp.concatenate([keys, pad_k], axis=0)
  dstp_ref[...] = jnp.concatenate([dst_ref[...], pad_d], axis=0)


def _prep_edges(src2d, dst2d, et2d):
  return pl.pallas_call(
      _prep_body,
      grid=(PGRID,),
      in_specs=[
          pl.BlockSpec((PB_IN, CHUNK), lambda i: (i, 0)),
          pl.BlockSpec((PB_IN, CHUNK), lambda i: (i, 0)),
          pl.BlockSpec((PB_IN, CHUNK), lambda i: (i, 0)),
      ],
      out_specs=[
          pl.BlockSpec((PB_OUT, CHUNK), lambda i: (i, 0)),
          pl.BlockSpec((PB_OUT, CHUNK), lambda i: (i, 0)),
      ],
      out_shape=[
          jax.ShapeDtypeStruct((CH_TOTAL, CHUNK), jnp.int32),
          jax.ShapeDtypeStruct((CH_TOTAL, CHUNK), jnp.int32),
      ],
  )(src2d, dst2d, et2d)


def _transform1_body(x_ref, wr_ref, t_ref):
  xb = x_ref[...]
  for r in range(R):
    t_ref[r] = jnp.dot(xb, wr_ref[r], preferred_element_type=jnp.float32)


def _transform1(x, Wr1):
  return pl.pallas_call(
      _transform1_body,
      grid=(GRID_N,),
      in_specs=[
          pl.BlockSpec((BN, D_IN), lambda i: (i, 0)),
          pl.BlockSpec((R, D_IN, D_H), lambda i: (0, 0, 0)),
      ],
      out_specs=pl.BlockSpec((R, BN, D_H), lambda i: (0, i, 0)),
      out_shape=jax.ShapeDtypeStruct((R, N, D_H), jnp.float32),
  )(x, Wr1)


def _mid_body(a_ref, b_ref, x_ref, ws_ref, bias_ref, wr_ref, h_ref, t_ref):
  agg = a_ref[0] + b_ref[0]
  h = jnp.tanh(agg + jnp.dot(x_ref[...], ws_ref[...],
                             preferred_element_type=jnp.float32)
               + bias_ref[...])
  h_ref[...] = h
  for r in range(R):
    t_ref[r] = jnp.dot(h, wr_ref[r], preferred_element_type=jnp.float32)


def _layer_mid(aggp, x, Ws1, b1, Wr2):
  return pl.pallas_call(
      _mid_body,
      grid=(GRID_N,),
      in_specs=[
          pl.BlockSpec((1, BN, D_H), lambda i: (0, i, 0)),
          pl.BlockSpec((1, BN, D_H), lambda i: (1, i, 0)),
          pl.BlockSpec((BN, D_IN), lambda i: (i, 0)),
          pl.BlockSpec((D_IN, D_H), lambda i: (0, 0)),
          pl.BlockSpec((1, D_H), lambda i: (0, 0)),
          pl.BlockSpec((R, D_H, D_H), lambda i: (0, 0, 0)),
      ],
      out_specs=[
          pl.BlockSpec((BN, D_H), lambda i: (i, 0)),
          pl.BlockSpec((R, BN, D_H), lambda i: (0, i, 0)),
      ],
      out_shape=[
          jax.ShapeDtypeStruct((N, D_H), jnp.float32),
          jax.ShapeDtypeStruct((R, N, D_H), jnp.float32),
      ],
  )(aggp, aggp, x, Ws1, b1, Wr2)


def _head_body(a_ref, b_ref, h1_ref, x_ref, ws_ref, bias_ref,
               wi1a_ref, wi1b_ref, bi1_ref, wi2_ref, bi2_ref,
               wj1a_ref, wj1b_ref, bj1_ref, wj2_ref, bj2_ref,
               wf1_ref, bf1_ref, wf2_ref, bf2_ref,
               out_ref, acc_ref):
  i = pl.program_id(0)

  @pl.when(i == 0)
  def _():
    acc_ref[...] = jnp.zeros_like(acc_ref)

  agg = a_ref[0] + b_ref[0]
  h1 = h1_ref[...]
  xb = x_ref[...]
  h2 = jnp.tanh(agg + jnp.dot(h1, ws_ref[...],
                              preferred_element_type=jnp.float32)
                + bias_ref[...])
  u = jnp.maximum(
      jnp.dot(xb, wi1a_ref[...], preferred_element_type=jnp.float32)
      + jnp.dot(h2, wi1b_ref[...], preferred_element_type=jnp.float32)
      + bi1_ref[...], 0.0)
  io = jax.nn.sigmoid(jnp.dot(u, wi2_ref[...],
                              preferred_element_type=jnp.float32)
                      + bi2_ref[...])
  v = jnp.maximum(
      jnp.dot(xb, wj1a_ref[...], preferred_element_type=jnp.float32)
      + jnp.dot(h2, wj1b_ref[...], preferred_element_type=jnp.float32)
      + bj1_ref[...], 0.0)
  jo = jnp.tanh(jnp.dot(v, wj2_ref[...], preferred_element_type=jnp.float32)
                + bj2_ref[...])
  p = jnp.sum(io * jo, axis=0, keepdims=True)
  acc_ref[0:1, 0:D_H] = acc_ref[0:1, 0:D_H] + p

  @pl.when(i == GRID_N - 1)
  def _():
    g = jnp.tanh(acc_ref[0:1, 0:D_H])
    f = jnp.maximum(
        jnp.dot(g, wf1_ref[...], preferred_element_type=jnp.float32)
        + bf1_ref[...], 0.0)
    out_ref[...] = (jnp.dot(f, wf2_ref[...],
                            preferred_element_type=jnp.float32)
                    + bf2_ref[...])


def _head(aggp, h1, x, Ws2, b2, Wi1a, Wi1b, bi1, Wi2, bi2,
          Wj1a, Wj1b, bj1, Wj2, bj2, Wf1, bf1, Wf2, bf2):
  full = lambda shape: pl.BlockSpec(shape, lambda i: tuple(0 for _ in shape))
  return pl.pallas_call(
      _head_body,
      grid=(GRID_N,),
      in_specs=[
          pl.BlockSpec((1, BN, D_H), lambda i: (0, i, 0)),
          pl.BlockSpec((1, BN, D_H), lambda i: (1, i, 0)),
          pl.BlockSpec((BN, D_H), lambda i: (i, 0)),
          pl.BlockSpec((BN, D_IN), lambda i: (i, 0)),
          full((D_H, D_H)), full((1, D_H)),
          full((D_IN, 64)), full((D_H, 64)), full((1, 64)),
          full((64, D_H)), full((1, D_H)),
          full((D_IN, 64)), full((D_H, 64)), full((1, 64)),
          full((64, D_H)), full((1, D_H)),
          full((D_H, 64)), full((1, 64)), full((64, 1)), full((1, 1)),
      ],
      out_specs=pl.BlockSpec((1, 1), lambda i: (0, 0)),
      out_shape=jax.ShapeDtypeStruct((1, 1), jnp.float32),
      scratch_shapes=[pltpu.VMEM((8, 128), jnp.float32)],
  )(aggp, aggp, h1, x, Ws2, b2, Wi1a, Wi1b, bi1, Wi2, bi2,
    Wj1a, Wj1b, bj1, Wj2, bj2, Wf1, bf1, Wf2, bf2)


def kernel(x, edge_index, edge_type, Wr1, Ws1, b1, Wr2, Ws2, b2,
           Wi1, bi1, Wi2, bi2, Wj1, bj1, Wj2, bj2, Wf1, bf1, Wf2, bf2):
  src2d = edge_index[0].reshape(EROWS_IN, CHUNK)
  dst2d = edge_index[1].reshape(EROWS_IN, CHUNK)
  et2d = edge_type.reshape(EROWS_IN, CHUNK)

  keys2d, dstp2d = _prep_edges(src2d, dst2d, et2d)
  zeros128 = jnp.zeros((CHUNK, D_H), jnp.float32)

  t1 = _transform1(x, Wr1)
  aggp1 = _sc_segsum(keys2d, dstp2d, t1.reshape(R * N, D_H), zeros128)
  h1, t2 = _layer_mid(aggp1.reshape(NC, N_ACC, D_H), x, Ws1,
                      b1.reshape(1, D_H), Wr2)
  aggp2 = _sc_segsum(keys2d, dstp2d, t2.reshape(R * N, D_H), zeros128)
  out = _head(aggp2.reshape(NC, N_ACC, D_H), h1, x, Ws2,
              b2.reshape(1, D_H),
              Wi1[:D_IN], Wi1[D_IN:], bi1.reshape(1, 64),
              Wi2, bi2.reshape(1, D_H),
              Wj1[:D_IN], Wj1[D_IN:], bj1.reshape(1, 64),
              Wj2, bj2.reshape(1, D_H),
              Wf1, bf1.reshape(1, 64), Wf2, bf2.reshape(1, 1))
  return out.reshape(1)


# speedup vs baseline: 20.2852x; 1.0643x over previous
import jax
import jax.numpy as jnp
from jax import lax
from jax.experimental import pallas as pl
from jax.experimental.pallas import tpu as pltpu
from jax.experimental.pallas import tpu_sc as plsc

N = 50000
E = 800000
R = 4
D_IN = 16
D_H = 32

NC = 2
NS = 16
NW = NC * NS
CHUNK = 128
E_PAD = 819200
CH_TOTAL = E_PAD // CHUNK
CH_PER_TILE = CH_TOTAL // NW
K = 5
SG = 20
NSG0 = 18
NSG1 = 2
N_ACC = 51200
ZROWS = N_ACC // NS
ZCH = ZROWS // CHUNK

EROWS_IN = E // CHUNK
PB_IN = EROWS_IN
PB_OUT = CH_TOTAL
PGRID = 1

BN = 2000
GRID_N = N // BN


def _sc_body(keys_hbm, dst_hbm, table_hbm, zeros_hbm, out_hbm,
             keysb, dstb, rows_v, agg_sh, gsem, ssem, zsem):
  c = lax.axis_index("c")
  s = lax.axis_index("s")
  tid = c * NS + s
  zbase = s * ZROWS

  pltpu.sync_copy(zeros_hbm, rows_v.at[0])

  def zgroup(i, carry):
    ds = []
    for b in range(K):
      r0 = pl.multiple_of(zbase + (i * K + b) * CHUNK, CHUNK)
      ds.append(pltpu.async_copy(rows_v.at[0], agg_sh.at[pl.ds(r0, CHUNK)],
                                 zsem))
    for d in ds:
      d.wait()
    return carry

  lax.fori_loop(0, ZCH // K, zgroup, 0)
  r0 = pl.multiple_of(zbase + (ZCH // K) * K * CHUNK, CHUNK)
  pltpu.sync_copy(rows_v.at[0], agg_sh.at[pl.ds(r0, CHUNK)])

  nsg = jnp.where(c == 0, NSG0, NSG1)
  cbase = pl.multiple_of(
      jnp.where(c == 0, s * (NSG0 * SG),
                16 * NSG0 * SG + s * (NSG1 * SG)), 8)
  plsc.subcore_barrier()

  def sg_body(sg, carry):
    i0 = pl.multiple_of(cbase + sg * SG, 8)
    pltpu.sync_copy(keys_hbm.at[pl.ds(i0, SG)], keysb)
    pltpu.sync_copy(dst_hbm.at[pl.ds(i0, SG)], dstb)

    def group_body(g, carry2):
      gds = [pltpu.async_copy(table_hbm.at[keysb.at[g * K + b]],
                              rows_v.at[b], gsem) for b in range(K)]
      for d in gds:
        d.wait()
      sds = [pltpu.async_copy(rows_v.at[b], agg_sh.at[dstb.at[g * K + b]],
                              ssem, add=True) for b in range(K)]
      for d in sds:
        d.wait()
      return carry2

    lax.fori_loop(0, SG // K, group_body, 0)
    return carry

  lax.fori_loop(0, nsg, sg_body, 0)
  plsc.subcore_barrier()

  obase = c * N_ACC + s * ZROWS

  def ogroup(i, carry):
    ds = []
    for b in range(K):
      r0 = pl.multiple_of(zbase + (i * K + b) * CHUNK, CHUNK)
      ds.append(pltpu.async_copy(agg_sh.at[pl.ds(r0, CHUNK)], rows_v.at[b],
                                 zsem))
    for d in ds:
      d.wait()
    ds = []
    for b in range(K):
      o0 = pl.multiple_of(obase + (i * K + b) * CHUNK, CHUNK)
      ds.append(pltpu.async_copy(rows_v.at[b], out_hbm.at[pl.ds(o0, CHUNK)],
                                 zsem))
    for d in ds:
      d.wait()
    return carry

  lax.fori_loop(0, ZCH // K, ogroup, 0)
  r0 = pl.multiple_of(zbase + (ZCH // K) * K * CHUNK, CHUNK)
  o0 = pl.multiple_of(obase + (ZCH // K) * K * CHUNK, CHUNK)
  pltpu.sync_copy(agg_sh.at[pl.ds(r0, CHUNK)], rows_v.at[0])
  pltpu.sync_copy(rows_v.at[0], out_hbm.at[pl.ds(o0, CHUNK)])


_sc_segsum = pl.kernel(
    _sc_body,
    out_type=jax.ShapeDtypeStruct((NC * N_ACC, D_H), jnp.float32),
    mesh=plsc.VectorSubcoreMesh(core_axis_name="c", subcore_axis_name="s"),
    scratch_types=[
        pltpu.VMEM((SG, CHUNK), jnp.int32),
        pltpu.VMEM((SG, CHUNK), jnp.int32),
        pltpu.VMEM((K, CHUNK, D_H), jnp.float32),
        pltpu.VMEM_SHARED((N_ACC, D_H), jnp.float32),
        pltpu.SemaphoreType.DMA,
        pltpu.SemaphoreType.DMA,
        pltpu.SemaphoreType.DMA,
    ],
    compiler_params=pltpu.CompilerParams(use_tc_tiling_on_sc=False),
)


def _prep_body(src_ref, dst_ref, et_ref, keys_ref, dstp_ref):
  keys = et_ref[...] * N + src_ref[...]
  pad_k = jnp.zeros((PB_OUT - PB_IN, CHUNK), jnp.int32)
  flat = jax.lax.broadcasted_iota(jnp.int32, (PB_OUT - PB_IN, CHUNK), 0) \
      * CHUNK + jax.lax.broadcasted_iota(
          jnp.int32, (PB_OUT - PB_IN, CHUNK), 1)
  pad_d = N + (flat % (N_ACC - N))
  keys_ref[...] = jnp.concatenate([keys, pad_k], axis=0)
  dstp_ref[...] = jnp.concatenate([dst_ref[...], pad_d], axis=0)


def _prep_edges(src2d, dst2d, et2d):
  return pl.pallas_call(
      _prep_body,
      grid=(PGRID,),
      in_specs=[
          pl.BlockSpec((PB_IN, CHUNK), lambda i: (i, 0)),
          pl.BlockSpec((PB_IN, CHUNK), lambda i: (i, 0)),
          pl.BlockSpec((PB_IN, CHUNK), lambda i: (i, 0)),
      ],
      out_specs=[
          pl.BlockSpec((PB_OUT, CHUNK), lambda i: (i, 0)),
          pl.BlockSpec((PB_OUT, CHUNK), lambda i: (i, 0)),
      ],
      out_shape=[
          jax.ShapeDtypeStruct((CH_TOTAL, CHUNK), jnp.int32),
          jax.ShapeDtypeStruct((CH_TOTAL, CHUNK), jnp.int32),
      ],
  )(src2d, dst2d, et2d)


def _transform1_body(x_ref, wr_ref, t_ref):
  xb = x_ref[...]
  for r in range(R):
    t_ref[r] = jnp.dot(xb, wr_ref[r], preferred_element_type=jnp.float32)


def _transform1(x, Wr1):
  return pl.pallas_call(
      _transform1_body,
      grid=(GRID_N,),
      in_specs=[
          pl.BlockSpec((BN, D_IN), lambda i: (i, 0)),
          pl.BlockSpec((R, D_IN, D_H), lambda i: (0, 0, 0)),
      ],
      out_specs=pl.BlockSpec((R, BN, D_H), lambda i: (0, i, 0)),
      out_shape=jax.ShapeDtypeStruct((R, N, D_H), jnp.float32),
  )(x, Wr1)


def _mid_body(a_ref, b_ref, x_ref, ws_ref, bias_ref, wr_ref, h_ref, t_ref):
  agg = a_ref[0] + b_ref[0]
  h = jnp.tanh(agg + jnp.dot(x_ref[...], ws_ref[...],
                             preferred_element_type=jnp.float32)
               + bias_ref[...])
  h_ref[...] = h
  for r in range(R):
    t_ref[r] = jnp.dot(h, wr_ref[r], preferred_element_type=jnp.float32)


def _layer_mid(aggp, x, Ws1, b1, Wr2):
  return pl.pallas_call(
      _mid_body,
      grid=(GRID_N,),
      in_specs=[
          pl.BlockSpec((1, BN, D_H), lambda i: (0, i, 0)),
          pl.BlockSpec((1, BN, D_H), lambda i: (1, i, 0)),
          pl.BlockSpec((BN, D_IN), lambda i: (i, 0)),
          pl.BlockSpec((D_IN, D_H), lambda i: (0, 0)),
          pl.BlockSpec((1, D_H), lambda i: (0, 0)),
          pl.BlockSpec((R, D_H, D_H), lambda i: (0, 0, 0)),
      ],
      out_specs=[
          pl.BlockSpec((BN, D_H), lambda i: (i, 0)),
          pl.BlockSpec((R, BN, D_H), lambda i: (0, i, 0)),
      ],
      out_shape=[
          jax.ShapeDtypeStruct((N, D_H), jnp.float32),
          jax.ShapeDtypeStruct((R, N, D_H), jnp.float32),
      ],
  )(aggp, aggp, x, Ws1, b1, Wr2)


def _head_body(a_ref, b_ref, h1_ref, x_ref, ws_ref, bias_ref,
               wi1a_ref, wi1b_ref, bi1_ref, wi2_ref, bi2_ref,
               wj1a_ref, wj1b_ref, bj1_ref, wj2_ref, bj2_ref,
               wf1_ref, bf1_ref, wf2_ref, bf2_ref,
               out_ref, acc_ref):
  i = pl.program_id(0)

  @pl.when(i == 0)
  def _():
    acc_ref[...] = jnp.zeros_like(acc_ref)

  agg = a_ref[0] + b_ref[0]
  h1 = h1_ref[...]
  xb = x_ref[...]
  h2 = jnp.tanh(agg + jnp.dot(h1, ws_ref[...],
                              preferred_element_type=jnp.float32)
                + bias_ref[...])
  u = jnp.maximum(
      jnp.dot(xb, wi1a_ref[...], preferred_element_type=jnp.float32)
      + jnp.dot(h2, wi1b_ref[...], preferred_element_type=jnp.float32)
      + bi1_ref[...], 0.0)
  io = jax.nn.sigmoid(jnp.dot(u, wi2_ref[...],
                              preferred_element_type=jnp.float32)
                      + bi2_ref[...])
  v = jnp.maximum(
      jnp.dot(xb, wj1a_ref[...], preferred_element_type=jnp.float32)
      + jnp.dot(h2, wj1b_ref[...], preferred_element_type=jnp.float32)
      + bj1_ref[...], 0.0)
  jo = jnp.tanh(jnp.dot(v, wj2_ref[...], preferred_element_type=jnp.float32)
                + bj2_ref[...])
  p = jnp.sum(io * jo, axis=0, keepdims=True)
  acc_ref[0:1, 0:D_H] = acc_ref[0:1, 0:D_H] + p

  @pl.when(i == GRID_N - 1)
  def _():
    g = jnp.tanh(acc_ref[0:1, 0:D_H])
    f = jnp.maximum(
        jnp.dot(g, wf1_ref[...], preferred_element_type=jnp.float32)
        + bf1_ref[...], 0.0)
    out_ref[...] = (jnp.dot(f, wf2_ref[...],
                            preferred_element_type=jnp.float32)
                    + bf2_ref[...])


def _head(aggp, h1, x, Ws2, b2, Wi1a, Wi1b, bi1, Wi2, bi2,
          Wj1a, Wj1b, bj1, Wj2, bj2, Wf1, bf1, Wf2, bf2):
  full = lambda shape: pl.BlockSpec(shape, lambda i: tuple(0 for _ in shape))
  return pl.pallas_call(
      _head_body,
      grid=(GRID_N,),
      in_specs=[
          pl.BlockSpec((1, BN, D_H), lambda i: (0, i, 0)),
          pl.BlockSpec((1, BN, D_H), lambda i: (1, i, 0)),
          pl.BlockSpec((BN, D_H), lambda i: (i, 0)),
          pl.BlockSpec((BN, D_IN), lambda i: (i, 0)),
          full((D_H, D_H)), full((1, D_H)),
          full((D_IN, 64)), full((D_H, 64)), full((1, 64)),
          full((64, D_H)), full((1, D_H)),
          full((D_IN, 64)), full((D_H, 64)), full((1, 64)),
          full((64, D_H)), full((1, D_H)),
          full((D_H, 64)), full((1, 64)), full((64, 1)), full((1, 1)),
      ],
      out_specs=pl.BlockSpec((1, 1), lambda i: (0, 0)),
      out_shape=jax.ShapeDtypeStruct((1, 1), jnp.float32),
      scratch_shapes=[pltpu.VMEM((8, 128), jnp.float32)],
  )(aggp, aggp, h1, x, Ws2, b2, Wi1a, Wi1b, bi1, Wi2, bi2,
    Wj1a, Wj1b, bj1, Wj2, bj2, Wf1, bf1, Wf2, bf2)


def kernel(x, edge_index, edge_type, Wr1, Ws1, b1, Wr2, Ws2, b2,
           Wi1, bi1, Wi2, bi2, Wj1, bj1, Wj2, bj2, Wf1, bf1, Wf2, bf2):
  src2d = edge_index[0].reshape(EROWS_IN, CHUNK)
  dst2d = edge_index[1].reshape(EROWS_IN, CHUNK)
  et2d = edge_type.reshape(EROWS_IN, CHUNK)

  keys2d, dstp2d = _prep_edges(src2d, dst2d, et2d)
  zeros128 = jnp.zeros((CHUNK, D_H), jnp.float32)

  t1 = _transform1(x, Wr1)
  aggp1 = _sc_segsum(keys2d, dstp2d, t1.reshape(R * N, D_H), zeros128)
  h1, t2 = _layer_mid(aggp1.reshape(NC, N_ACC, D_H), x, Ws1,
                      b1.reshape(1, D_H), Wr2)
  aggp2 = _sc_segsum(keys2d, dstp2d, t2.reshape(R * N, D_H), zeros128)
  out = _head(aggp2.reshape(NC, N_ACC, D_H), h1, x, Ws2,
              b2.reshape(1, D_H),
              Wi1[:D_IN], Wi1[D_IN:], bi1.reshape(1, 64),
              Wi2, bi2.reshape(1, D_H),
              Wj1[:D_IN], Wj1[D_IN:], bj1.reshape(1, 64),
              Wj2, bj2.reshape(1, D_H),
              Wf1, bf1.reshape(1, 64), Wf2, bf2.reshape(1, 1))
  return out.reshape(1)
